# baseline (speedup 1.0000x reference)
import functools

import jax
import jax.numpy as jnp
from jax import lax
from jax.experimental import pallas as pl
from jax.experimental.pallas import tpu as pltpu
from jax.experimental.pallas import tpu_sc as plsc

BATCH = 16384
EMBED = 32
NUM_CORES = 2
NUM_SUBCORES = 16
NW = NUM_CORES * NUM_SUBCORES
BPW = BATCH // NW
CH = 128
NCH = BPW // CH
LANES = 16
GRP = CH // LANES


def _mf_body(user_ref, item_ref, uef_ref, ief_ref, ub_ref, ib_ref, out_ref,
             idx_u, idx_i, ue_t, ie_t, ub_v, ib_v, out_v, sem):
    wid = lax.axis_index("s") * NUM_CORES + lax.axis_index("c")
    base = wid * BPW

    cps = []
    for j in range(NCH):
        cps.append(pltpu.async_copy(
            user_ref.at[pl.ds(base + j * CH, CH)], idx_u.at[j], sem))
        cps.append(pltpu.async_copy(
            item_ref.at[pl.ds(base + j * CH, CH)], idx_i.at[j], sem))
    for c in cps:
        c.wait()

    for j in range(NCH):
        cps = [
            pltpu.async_copy(ub_ref.at[idx_u.at[j]], ub_v.at[j], sem),
            pltpu.async_copy(ib_ref.at[idx_i.at[j]], ib_v.at[j], sem),
        ]
        for c in range(EMBED):
            cps.append(pltpu.async_copy(
                uef_ref.at[c].at[idx_u.at[j]], ue_t.at[c], sem))
            cps.append(pltpu.async_copy(
                ief_ref.at[c].at[idx_i.at[j]], ie_t.at[c], sem))
        for c in cps:
            c.wait()

        for t in range(GRP):
            sl = pl.ds(t * LANES, LANES)
            acc = ub_v[j, sl] + ib_v[j, sl]
            for c in range(EMBED):
                acc = acc + ue_t[c, sl] * ie_t[c, sl]
            out_v[sl] = acc

        pltpu.sync_copy(out_v, out_ref.at[pl.ds(base + j * CH, CH)])


@functools.partial(jax.jit, static_argnames=())
def kernel(user, item, user_e, item_e, user_b, item_b):
    mesh = plsc.VectorSubcoreMesh(core_axis_name="c", subcore_axis_name="s")
    k = pl.kernel(
        _mf_body,
        out_type=jax.ShapeDtypeStruct((BATCH,), jnp.float32),
        mesh=mesh,
        compiler_params=pltpu.CompilerParams(
            needs_layout_passes=False, use_tc_tiling_on_sc=False),
        scratch_types=[
            pltpu.VMEM((NCH, CH), jnp.int32),
            pltpu.VMEM((NCH, CH), jnp.int32),
            pltpu.VMEM((EMBED, CH), jnp.float32),
            pltpu.VMEM((EMBED, CH), jnp.float32),
            pltpu.VMEM((NCH, CH), jnp.float32),
            pltpu.VMEM((NCH, CH), jnp.float32),
            pltpu.VMEM((CH,), jnp.float32),
            pltpu.SemaphoreType.DMA,
        ],
    )
    return k(user.astype(jnp.int32), item.astype(jnp.int32),
             user_e.T, item_e.T, user_b.reshape(-1), item_b.reshape(-1))


# speedup vs baseline: 4.7600x; 4.7600x over previous
import functools

import jax
import jax.numpy as jnp
from jax import lax
from jax.experimental import pallas as pl
from jax.experimental.pallas import tpu as pltpu
from jax.experimental.pallas import tpu_sc as plsc

BATCH = 16384
EMBED = 32
NUM_CORES = 2
NUM_SUBCORES = 16
NW = NUM_CORES * NUM_SUBCORES
BPW = BATCH // NW
CH = 128
NCH = BPW // CH
LANES = 16
GROUPS = BPW // LANES


def _mf_body(user_ref, item_ref, ue_ref, ie_ref, ub_ref, ib_ref, out_ref,
             idx_u, idx_i, ue_v, ie_v, ub_v, ib_v, out_v, sem):
    wid = lax.axis_index("s") * NUM_CORES + lax.axis_index("c")
    base = wid * BPW

    cps = []
    for j in range(NCH):
        cps.append(pltpu.async_copy(
            user_ref.at[pl.ds(base + j * CH, CH)], idx_u.at[j], sem))
        cps.append(pltpu.async_copy(
            item_ref.at[pl.ds(base + j * CH, CH)], idx_i.at[j], sem))
    for c in cps:
        c.wait()

    cps = []
    for j in range(NCH):
        cps.append(pltpu.async_copy(
            ue_ref.at[idx_u.at[j]], ue_v.at[pl.ds(j * CH, CH)], sem))
        cps.append(pltpu.async_copy(
            ie_ref.at[idx_i.at[j]], ie_v.at[pl.ds(j * CH, CH)], sem))
        cps.append(pltpu.async_copy(ub_ref.at[idx_u.at[j]], ub_v.at[j], sem))
        cps.append(pltpu.async_copy(ib_ref.at[idx_i.at[j]], ib_v.at[j], sem))
    for c in cps:
        c.wait()

    iota = lax.iota(jnp.int32, LANES)
    for g in range(GROUPS):
        rows = iota + g * LANES
        jg, off = divmod(g * LANES, CH)
        acc = ub_v[jg, pl.ds(off, LANES)] + ib_v[jg, pl.ds(off, LANES)]
        for d in range(EMBED):
            col = jnp.full((LANES,), d, jnp.int32)
            acc = acc + (plsc.load_gather(ue_v, [rows, col])
                         * plsc.load_gather(ie_v, [rows, col]))
        out_v[pl.ds(g * LANES, LANES)] = acc

    pltpu.sync_copy(out_v, out_ref.at[pl.ds(base, BPW)])


@functools.partial(jax.jit, static_argnames=())
def kernel(user, item, user_e, item_e, user_b, item_b):
    mesh = plsc.VectorSubcoreMesh(core_axis_name="c", subcore_axis_name="s")
    k = pl.kernel(
        _mf_body,
        out_type=jax.ShapeDtypeStruct((BATCH,), jnp.float32),
        mesh=mesh,
        compiler_params=pltpu.CompilerParams(
            needs_layout_passes=False, use_tc_tiling_on_sc=False,
            skip_device_barrier=True),
        scratch_types=[
            pltpu.VMEM((NCH, CH), jnp.int32),
            pltpu.VMEM((NCH, CH), jnp.int32),
            pltpu.VMEM((BPW, EMBED), jnp.float32),
            pltpu.VMEM((BPW, EMBED), jnp.float32),
            pltpu.VMEM((NCH, CH), jnp.float32),
            pltpu.VMEM((NCH, CH), jnp.float32),
            pltpu.VMEM((BPW,), jnp.float32),
            pltpu.SemaphoreType.DMA,
        ],
    )
    return k(user.astype(jnp.int32), item.astype(jnp.int32),
             user_e, item_e, user_b.reshape(-1), item_b.reshape(-1))
